# Initial kernel scaffold; baseline (speedup 1.0000x reference)
import jax
import jax.numpy as jnp
from jax.experimental import pallas as pl


def kernel(x, table):
    raise NotImplementedError("write your pallas kernel here")


# speedup vs baseline: 51.5728x; 51.5728x over previous
import functools

import jax
import jax.numpy as jnp
from jax import lax
from jax.experimental import pallas as pl
from jax.experimental.pallas import tpu as pltpu
from jax.experimental.pallas import tpu_sc as plsc

N_POINTS = 1048576
IN_DIM = 3
N_FEATS = 8
HASHMAP_SIZE = 524288
HASH_MASK = HASHMAP_SIZE - 1
RES = 512.0
PRIME1 = -1640531535
PRIME2 = 805459861

NC = 2
NS = 16
NW = NC * NS
NP = N_POINTS // NW
C = 512
G = NP // C
NIDX = N_FEATS * C
IROWS = NIDX // 128


def _body(x_hbm, table_hbm, out_hbm, xbuf, idxbuf, wbuf, rowsbuf, outbuf, gsem):
    wid = lax.axis_index("s") * NC + lax.axis_index("c")
    base_pt = wid * NP
    i16 = lax.iota(jnp.int32, 16)
    fcol = [jnp.full((16,), f, jnp.int32) for f in range(N_FEATS)]
    dcol = [jnp.full((16,), d, jnp.int32) for d in range(IN_DIM)]

    @pl.loop(0, G)
    def _chunk(g):
        pbase = base_pt + g * C
        pltpu.sync_copy(x_hbm.at[pl.ds(pbase, C)], xbuf)

        @pl.loop(0, C // 16)
        def _p1(t):
            pt = t * 16 + i16
            h0, h1, xf, om = [], [], [], []
            for d in range(IN_DIM):
                xs = plsc.load_gather(xbuf, [pt, dcol[d]]) * RES
                xi = xs.astype(jnp.int32)
                frac = xs - xi.astype(jnp.float32)
                xf.append(frac)
                om.append(1.0 - frac)
                if d == 0:
                    h0.append(xi)
                    h1.append(xi + 1)
                else:
                    p = PRIME1 if d == 1 else PRIME2
                    hp = xi * p
                    h0.append(hp)
                    h1.append(hp + p)
            for j in range(1 << IN_DIM):
                hid = None
                w = None
                for d in range(IN_DIM):
                    bit = (j >> d) & 1
                    hd = h1[d] if bit else h0[d]
                    wd = xf[d] if bit else om[d]
                    hid = hd if hid is None else hid ^ hd
                    w = wd if w is None else w * wd
                hid = hid & HASH_MASK
                flat = j * C + t * 16
                idxbuf[flat >> 7, pl.ds(flat & 127, 16)] = hid
                wbuf[pl.ds(flat, 16)] = w

        @pl.loop(0, IROWS)
        def _fire(k):
            pltpu.async_copy(
                table_hbm.at[idxbuf.at[k]],
                rowsbuf.at[pl.ds(k * 128, 128)],
                gsem,
            )

        pltpu.make_async_copy(
            table_hbm.at[pl.ds(0, NIDX)], rowsbuf, gsem
        ).wait()

        @pl.loop(0, C // 16)
        def _p3(t):
            pt = t * 16 + i16
            accs = [None] * N_FEATS
            for j in range(1 << IN_DIM):
                flat = j * C + t * 16
                wv = wbuf[pl.ds(flat, 16)]
                rid = flat + i16
                for f in range(N_FEATS):
                    rv = plsc.load_gather(rowsbuf, [rid, fcol[f]])
                    term = rv * wv
                    accs[f] = term if accs[f] is None else accs[f] + term
            for f in range(N_FEATS):
                plsc.store_scatter(outbuf, [pt, fcol[f]], accs[f])

        pltpu.sync_copy(outbuf, out_hbm.at[pl.ds(pbase, C)])


@functools.partial(
    pl.kernel,
    out_type=jax.ShapeDtypeStruct((N_POINTS, N_FEATS), jnp.float32),
    mesh=plsc.VectorSubcoreMesh(
        core_axis_name="c", subcore_axis_name="s", num_cores=NC, num_subcores=NS
    ),
    compiler_params=pltpu.CompilerParams(
        needs_layout_passes=False, use_tc_tiling_on_sc=False
    ),
    scratch_types=[
        pltpu.VMEM((C, IN_DIM), jnp.float32),
        pltpu.VMEM((IROWS, 128), jnp.int32),
        pltpu.VMEM((NIDX,), jnp.float32),
        pltpu.VMEM((NIDX, N_FEATS), jnp.float32),
        pltpu.VMEM((C, N_FEATS), jnp.float32),
        pltpu.SemaphoreType.DMA,
    ],
)
def _hash_grid(x_hbm, table_hbm, out_hbm, xbuf, idxbuf, wbuf, rowsbuf, outbuf, gsem):
    _body(x_hbm, table_hbm, out_hbm, xbuf, idxbuf, wbuf, rowsbuf, outbuf, gsem)


def kernel(x, table):
    return _hash_grid(x, table)


# speedup vs baseline: 57.4385x; 1.1137x over previous
import functools

import jax
import jax.numpy as jnp
from jax import lax
from jax.experimental import pallas as pl
from jax.experimental.pallas import tpu as pltpu
from jax.experimental.pallas import tpu_sc as plsc

N_POINTS = 1048576
IN_DIM = 3
N_FEATS = 8
HASHMAP_SIZE = 524288
HASH_MASK = HASHMAP_SIZE - 1
RES = 512.0
PRIME1 = -1640531535
PRIME2 = 805459861

NC = 2
NS = 16
NW = NC * NS
NP = N_POINTS // NW
C = 512
G = NP // C
NIDX = N_FEATS * C
IROWS = NIDX // 128


def _body(x_hbm, table_hbm, out_hbm, xbuf, idxbuf, wbuf, rowsbuf, outbuf, gsem):
    wid = lax.axis_index("s") * NC + lax.axis_index("c")
    base_pt = wid * NP
    i16 = lax.iota(jnp.int32, 16)
    fcol = [jnp.full((16,), f, jnp.int32) for f in range(N_FEATS)]

    @pl.loop(0, G)
    def _chunk(g):
        pbase = base_pt + g * C
        pltpu.sync_copy(x_hbm.at[pl.ds(pbase * IN_DIM, C * IN_DIM)], xbuf)

        @pl.loop(0, C // 16)
        def _p1(t):
            pt3 = (t * 48) + i16 * 3
            h0, h1, xf, om = [], [], [], []
            for d in range(IN_DIM):
                xs = plsc.load_gather(xbuf, [pt3 + d]) * RES
                xi = xs.astype(jnp.int32)
                frac = xs - xi.astype(jnp.float32)
                xf.append(frac)
                om.append(1.0 - frac)
                if d == 0:
                    h0.append(xi)
                    h1.append(xi + 1)
                else:
                    p = PRIME1 if d == 1 else PRIME2
                    hp = xi * p
                    h0.append(hp)
                    h1.append(hp + p)
            for j in range(1 << IN_DIM):
                hid = None
                w = None
                for d in range(IN_DIM):
                    bit = (j >> d) & 1
                    hd = h1[d] if bit else h0[d]
                    wd = xf[d] if bit else om[d]
                    hid = hd if hid is None else hid ^ hd
                    w = wd if w is None else w * wd
                hid = hid & HASH_MASK
                flat = j * C + t * 16
                idxbuf[flat >> 7, pl.ds(flat & 127, 16)] = hid
                wbuf[pl.ds(flat, 16)] = w

        @pl.loop(0, IROWS)
        def _fire(k):
            pltpu.async_copy(
                table_hbm.at[idxbuf.at[k]],
                rowsbuf.at[pl.ds(k * 128, 128)],
                gsem,
            )

        pltpu.make_async_copy(
            table_hbm.at[pl.ds(0, NIDX)], rowsbuf, gsem
        ).wait()

        @pl.loop(0, C // 16)
        def _p3(t):
            pt8 = t * 128 + i16 * 8
            accs = [None] * N_FEATS
            for j in range(1 << IN_DIM):
                flat = j * C + t * 16
                wv = wbuf[pl.ds(flat, 16)]
                rid = flat + i16
                for f in range(N_FEATS):
                    rv = plsc.load_gather(rowsbuf, [rid, fcol[f]])
                    term = rv * wv
                    accs[f] = term if accs[f] is None else accs[f] + term
            for f in range(N_FEATS):
                plsc.store_scatter(outbuf, [pt8 + f], accs[f])

        pltpu.sync_copy(outbuf, out_hbm.at[pl.ds(pbase * N_FEATS, C * N_FEATS)])


@functools.partial(
    pl.kernel,
    out_type=jax.ShapeDtypeStruct((N_POINTS * N_FEATS,), jnp.float32),
    mesh=plsc.VectorSubcoreMesh(
        core_axis_name="c", subcore_axis_name="s", num_cores=NC, num_subcores=NS
    ),
    compiler_params=pltpu.CompilerParams(
        needs_layout_passes=False, use_tc_tiling_on_sc=False
    ),
    scratch_types=[
        pltpu.VMEM((C * IN_DIM,), jnp.float32),
        pltpu.VMEM((IROWS, 128), jnp.int32),
        pltpu.VMEM((NIDX,), jnp.float32),
        pltpu.VMEM((NIDX, N_FEATS), jnp.float32),
        pltpu.VMEM((C * N_FEATS,), jnp.float32),
        pltpu.SemaphoreType.DMA,
    ],
)
def _hash_grid(x_hbm, table_hbm, out_hbm, xbuf, idxbuf, wbuf, rowsbuf, outbuf, gsem):
    _body(x_hbm, table_hbm, out_hbm, xbuf, idxbuf, wbuf, rowsbuf, outbuf, gsem)


def kernel(x, table):
    out_flat = _hash_grid(x.reshape(-1), table)
    return out_flat.reshape(N_POINTS, N_FEATS)


# speedup vs baseline: 183.5639x; 3.1958x over previous
import functools

import jax
import jax.numpy as jnp
from jax import lax
from jax.experimental import pallas as pl
from jax.experimental.pallas import tpu as pltpu
from jax.experimental.pallas import tpu_sc as plsc

N_POINTS = 1048576
IN_DIM = 3
N_FEATS = 8
HASHMAP_SIZE = 524288
HASH_MASK = HASHMAP_SIZE - 1
RES = 512.0
PRIME1 = -1640531535
PRIME2 = 805459861

NC = 2
NS = 16
NW = NC * NS
NP = N_POINTS // NW
C = 512
G = NP // C
NIDX = N_FEATS * C
IROWS = NIDX // 128
XW = 4


def _body(x_hbm, table_hbm, out_hbm, xbuf, idxbuf, wbuf, rowsbuf, outbuf, gsem):
    wid = lax.axis_index("s") * NC + lax.axis_index("c")
    base_pt = wid * NP
    i16 = lax.iota(jnp.int32, 16)
    fcol = [jnp.full((16,), f, jnp.int32) for f in range(N_FEATS)]

    @pl.loop(0, G)
    def _chunk(g):
        pbase = base_pt + g * C
        pltpu.sync_copy(x_hbm.at[pl.ds(pbase * XW, C * XW)], xbuf)

        @pl.loop(0, C // 16)
        def _p1(t):
            xoff = (t >> 3) * (128 * XW) + (t & 7) * 16
            h0, h1, xf, om = [], [], [], []
            for d in range(IN_DIM):
                xs = xbuf[pl.ds(xoff + d * 128, 16)] * RES
                xi = xs.astype(jnp.int32)
                frac = xs - xi.astype(jnp.float32)
                xf.append(frac)
                om.append(1.0 - frac)
                if d == 0:
                    h0.append(xi)
                    h1.append(xi + 1)
                else:
                    p = PRIME1 if d == 1 else PRIME2
                    hp = xi * p
                    h0.append(hp)
                    h1.append(hp + p)
            for j in range(1 << IN_DIM):
                hid = None
                w = None
                for d in range(IN_DIM):
                    bit = (j >> d) & 1
                    hd = h1[d] if bit else h0[d]
                    wd = xf[d] if bit else om[d]
                    hid = hd if hid is None else hid ^ hd
                    w = wd if w is None else w * wd
                hid = hid & HASH_MASK
                flat = j * C + t * 16
                idxbuf[flat >> 7, pl.ds(flat & 127, 16)] = hid
                wbuf[pl.ds(flat, 16)] = w

        @pl.loop(0, IROWS)
        def _fire(k):
            pltpu.async_copy(
                table_hbm.at[idxbuf.at[k]],
                rowsbuf.at[pl.ds(k * 128, 128)],
                gsem,
            )

        pltpu.make_async_copy(
            table_hbm.at[pl.ds(0, NIDX)], rowsbuf, gsem
        ).wait()

        @pl.loop(0, C // 16)
        def _p3(t):
            ooff = (t >> 3) * (128 * N_FEATS) + (t & 7) * 16
            accs = [None] * N_FEATS
            for j in range(1 << IN_DIM):
                flat = j * C + t * 16
                wv = wbuf[pl.ds(flat, 16)]
                rid = flat + i16
                for f in range(N_FEATS):
                    rv = plsc.load_gather(rowsbuf, [rid, fcol[f]])
                    term = rv * wv
                    accs[f] = term if accs[f] is None else accs[f] + term
            for f in range(N_FEATS):
                outbuf[pl.ds(ooff + f * 128, 16)] = accs[f]

        pltpu.sync_copy(outbuf, out_hbm.at[pl.ds(pbase * N_FEATS, C * N_FEATS)])


@functools.partial(
    pl.kernel,
    out_type=jax.ShapeDtypeStruct((N_POINTS * N_FEATS,), jnp.float32),
    mesh=plsc.VectorSubcoreMesh(
        core_axis_name="c", subcore_axis_name="s", num_cores=NC, num_subcores=NS
    ),
    compiler_params=pltpu.CompilerParams(
        needs_layout_passes=False, use_tc_tiling_on_sc=False
    ),
    scratch_types=[
        pltpu.VMEM((C * XW,), jnp.float32),
        pltpu.VMEM((IROWS, 128), jnp.int32),
        pltpu.VMEM((NIDX,), jnp.float32),
        pltpu.VMEM((NIDX, N_FEATS), jnp.float32),
        pltpu.VMEM((C * N_FEATS,), jnp.float32),
        pltpu.SemaphoreType.DMA,
    ],
)
def _hash_grid(x_hbm, table_hbm, out_hbm, xbuf, idxbuf, wbuf, rowsbuf, outbuf, gsem):
    _body(x_hbm, table_hbm, out_hbm, xbuf, idxbuf, wbuf, rowsbuf, outbuf, gsem)


def kernel(x, table):
    xp = jnp.pad(x, ((0, 0), (0, XW - IN_DIM)))
    x_flat = xp.reshape(N_POINTS // 128, 128, XW).transpose(0, 2, 1).reshape(-1)
    out_flat = _hash_grid(x_flat, table)
    out = (
        out_flat.reshape(N_POINTS // 128, N_FEATS, 128)
        .transpose(0, 2, 1)
        .reshape(N_POINTS, N_FEATS)
    )
    return out


# speedup vs baseline: 274.8804x; 1.4975x over previous
import functools

import jax
import jax.numpy as jnp
from jax import lax
from jax.experimental import pallas as pl
from jax.experimental.pallas import tpu as pltpu
from jax.experimental.pallas import tpu_sc as plsc

N_POINTS = 1048576
IN_DIM = 3
N_FEATS = 8
HASHMAP_SIZE = 524288
HASH_MASK = HASHMAP_SIZE - 1
RES = 512.0
PRIME1 = -1640531535
PRIME2 = 805459861

NC = 2
NS = 16
NW = NC * NS
NP = N_POINTS // NW
C = 512
G = NP // C
NIDX = N_FEATS * C
IROWS = NIDX // 128
XW = 4


def _body(x_hbm, table_hbm, out_hbm, xb, ib, wb, rb, ob, gsem, osem):
    wid = lax.axis_index("s") * NC + lax.axis_index("c")
    base_pt = wid * NP
    i16 = lax.iota(jnp.int32, 16)
    fcol = [jnp.full((16,), f, jnp.int32) for f in range(N_FEATS)]

    def stage(g, par):
        pbase = base_pt + g * C
        xbuf, idxbuf, wbuf = xb[par], ib[par], wb[par]
        pltpu.sync_copy(x_hbm.at[pl.ds(pbase * XW, C * XW)], xbuf)

        @pl.loop(0, C // 16)
        def _p1(t):
            xoff = (t >> 3) * (128 * XW) + (t & 7) * 16
            h0, h1, xf, om = [], [], [], []
            for d in range(IN_DIM):
                xs = xbuf[pl.ds(xoff + d * 128, 16)] * RES
                xi = xs.astype(jnp.int32)
                frac = xs - xi.astype(jnp.float32)
                xf.append(frac)
                om.append(1.0 - frac)
                if d == 0:
                    h0.append(xi)
                    h1.append(xi + 1)
                else:
                    p = PRIME1 if d == 1 else PRIME2
                    hp = xi * p
                    h0.append(hp)
                    h1.append(hp + p)
            for j in range(1 << IN_DIM):
                hid = None
                w = None
                for d in range(IN_DIM):
                    bit = (j >> d) & 1
                    hd = h1[d] if bit else h0[d]
                    wd = xf[d] if bit else om[d]
                    hid = hd if hid is None else hid ^ hd
                    w = wd if w is None else w * wd
                hid = hid & HASH_MASK
                flat = j * C + t * 16
                idxbuf[flat >> 7, pl.ds(flat & 127, 16)] = hid
                wbuf[pl.ds(flat, 16)] = w

        @pl.loop(0, IROWS)
        def _fire(k):
            pltpu.async_copy(
                table_hbm.at[idxbuf.at[k]],
                rb[par].at[pl.ds(k * 128, 128)],
                gsem[par],
            )

    def combine(g, par):
        pbase = base_pt + g * C
        wbuf, rowsbuf, outbuf = wb[par], rb[par], ob[par]
        pltpu.make_async_copy(
            table_hbm.at[pl.ds(0, NIDX)], rowsbuf, gsem[par]
        ).wait()

        @pl.when(g >= 2)
        def _():
            pltpu.make_async_copy(
                x_hbm.at[pl.ds(0, C * N_FEATS)], outbuf, osem[par]
            ).wait()

        @pl.loop(0, C // 16)
        def _p3(t):
            ooff = (t >> 3) * (128 * N_FEATS) + (t & 7) * 16
            accs = [None] * N_FEATS
            for j in range(1 << IN_DIM):
                flat = j * C + t * 16
                wv = wbuf[pl.ds(flat, 16)]
                rid = flat + i16
                for f in range(N_FEATS):
                    rv = plsc.load_gather(rowsbuf, [rid, fcol[f]])
                    term = rv * wv
                    accs[f] = term if accs[f] is None else accs[f] + term
            for f in range(N_FEATS):
                outbuf[pl.ds(ooff + f * 128, 16)] = accs[f]

        pltpu.async_copy(
            outbuf, out_hbm.at[pl.ds(pbase * N_FEATS, C * N_FEATS)], osem[par]
        )

    stage(0, 0)

    @pl.loop(0, G // 2)
    def _gg(gg):
        g0 = 2 * gg
        stage(g0 + 1, 1)
        combine(g0, 0)

        @pl.when(g0 + 2 < G)
        def _():
            stage(g0 + 2, 0)

        combine(g0 + 1, 1)

    pltpu.make_async_copy(x_hbm.at[pl.ds(0, C * N_FEATS)], ob[0], osem[0]).wait()
    pltpu.make_async_copy(x_hbm.at[pl.ds(0, C * N_FEATS)], ob[1], osem[1]).wait()


@functools.partial(
    pl.kernel,
    out_type=jax.ShapeDtypeStruct((N_POINTS * N_FEATS,), jnp.float32),
    mesh=plsc.VectorSubcoreMesh(
        core_axis_name="c", subcore_axis_name="s", num_cores=NC, num_subcores=NS
    ),
    compiler_params=pltpu.CompilerParams(
        needs_layout_passes=False, use_tc_tiling_on_sc=False
    ),
    scratch_types=[
        [pltpu.VMEM((C * XW,), jnp.float32)] * 2,
        [pltpu.VMEM((IROWS, 128), jnp.int32)] * 2,
        [pltpu.VMEM((NIDX,), jnp.float32)] * 2,
        [pltpu.VMEM((NIDX, N_FEATS), jnp.float32)] * 2,
        [pltpu.VMEM((C * N_FEATS,), jnp.float32)] * 2,
        [pltpu.SemaphoreType.DMA] * 2,
        [pltpu.SemaphoreType.DMA] * 2,
    ],
)
def _hash_grid(x_hbm, table_hbm, out_hbm, xb, ib, wb, rb, ob, gsem, osem):
    _body(x_hbm, table_hbm, out_hbm, xb, ib, wb, rb, ob, gsem, osem)


def kernel(x, table):
    xp = jnp.pad(x, ((0, 0), (0, XW - IN_DIM)))
    x_flat = xp.reshape(N_POINTS // 128, 128, XW).transpose(0, 2, 1).reshape(-1)
    out_flat = _hash_grid(x_flat, table)
    out = (
        out_flat.reshape(N_POINTS // 128, N_FEATS, 128)
        .transpose(0, 2, 1)
        .reshape(N_POINTS, N_FEATS)
    )
    return out
